# Initial kernel scaffold; baseline (speedup 1.0000x reference)
import jax
import jax.numpy as jnp
from jax.experimental import pallas as pl


def kernel(x, table):
    raise NotImplementedError("write your pallas kernel here")


# speedup vs baseline: 1.2803x; 1.2803x over previous
import jax
import jax.numpy as jnp
from jax import lax
from jax.experimental import pallas as pl
from jax.experimental.pallas import tpu as pltpu
from jax.experimental.pallas import tpu_sc as plsc

D_MODEL = 128
NC = 2
NS = 16
NW = NC * NS

G = 128
NCHUNK = 200


def _emb_body(x_hbm, table_hbm, out_hbm, idx_v, rows_v, sem):
    c = lax.axis_index("c")
    s = lax.axis_index("s")
    wid = s * NC + c
    pltpu.sync_copy(x_hbm.at[wid], idx_v)
    base = wid * (NCHUNK * G)

    def chunk(i, carry):
        pltpu.async_copy(table_hbm.at[idx_v.at[i]], rows_v, sem).wait()
        pltpu.sync_copy(rows_v, out_hbm.at[pl.ds(base + i * G, G)])
        return carry

    lax.fori_loop(0, NCHUNK, chunk, 0)


@jax.jit
def _emb(xf, table):
    mesh = plsc.VectorSubcoreMesh(core_axis_name="c", subcore_axis_name="s")
    kern = pl.kernel(
        _emb_body,
        out_type=jax.ShapeDtypeStruct((NW * NCHUNK * G, D_MODEL), jnp.float32),
        mesh=mesh,
        scratch_types=[
            pltpu.VMEM((NCHUNK, G), jnp.int32),
            pltpu.VMEM((G, D_MODEL), jnp.float32),
            pltpu.SemaphoreType.DMA,
        ],
    )
    return kern(xf, table)


def kernel(x, table):
    b, t = x.shape
    xf = x.reshape(NW, NCHUNK, G).astype(jnp.int32)
    out = _emb(xf, table)
    return out.reshape(b, t, D_MODEL)


# speedup vs baseline: 1.8585x; 1.4517x over previous
import jax
import jax.numpy as jnp
from jax import lax
from jax.experimental import pallas as pl
from jax.experimental.pallas import tpu as pltpu
from jax.experimental.pallas import tpu_sc as plsc

D_MODEL = 128
NC = 2
NS = 16
NW = NC * NS

G = 128
NCHUNK = 200
NB = 4


def _emb_body(x_hbm, table_hbm, out_hbm, idx_v,
              r0, r1, r2, r3, g0, g1, g2, g3, w0, w1, w2, w3):
    rows = (r0, r1, r2, r3)
    gsem = (g0, g1, g2, g3)
    wsem = (w0, w1, w2, w3)
    cc = lax.axis_index("c")
    ss = lax.axis_index("s")
    wid = ss * NC + cc
    pltpu.sync_copy(x_hbm.at[wid], idx_v)
    base = wid * (NCHUNK * G)

    def fire_g(ci, b):
        pltpu.async_copy(table_hbm.at[idx_v.at[ci]], rows[b], gsem[b])

    def wait_g(b):
        pltpu.make_async_copy(table_hbm.at[pl.ds(0, G)], rows[b], gsem[b]).wait()

    def fire_w(ci, b):
        pltpu.async_copy(rows[b], out_hbm.at[pl.ds(base + ci * G, G)], wsem[b])

    def wait_w(b):
        pltpu.make_async_copy(rows[b], out_hbm.at[pl.ds(base, G)], wsem[b]).wait()

    fire_g(0, 0)
    fire_g(1, 1)
    fire_g(2, 2)

    wait_g(0)
    fire_w(0, 0)
    fire_g(3, 3)
    for b in range(1, NB):
        wait_g(b)
        fire_w(b, b)
        fb = (b + 3) % NB
        wait_w(fb)
        fire_g(b + 3, fb)

    def step(s, carry):
        for b in range(NB):
            ci = s * NB + b
            wait_g(b)
            fire_w(ci, b)
            fb = (b + 3) % NB
            wait_w(fb)
            fire_g(ci + 3, fb)
        return carry

    lax.fori_loop(1, NCHUNK // NB - 1, step, 0)

    s = NCHUNK // NB - 1
    for b in range(NB):
        ci = s * NB + b
        wait_g(b)
        fire_w(ci, b)
        fb = (b + 3) % NB
        wait_w(fb)
        if ci + 3 < NCHUNK:
            fire_g(ci + 3, fb)
    wait_w(3)


@jax.jit
def _emb(xf, table):
    mesh = plsc.VectorSubcoreMesh(core_axis_name="c", subcore_axis_name="s")
    kern = pl.kernel(
        _emb_body,
        out_type=jax.ShapeDtypeStruct((NW * NCHUNK * G, D_MODEL), jnp.float32),
        mesh=mesh,
        scratch_types=(
            [pltpu.VMEM((NCHUNK, G), jnp.int32)]
            + [pltpu.VMEM((G, D_MODEL), jnp.float32) for _ in range(NB)]
            + [pltpu.SemaphoreType.DMA for _ in range(2 * NB)]
        ),
    )
    return kern(xf, table)


def kernel(x, table):
    b, t = x.shape
    xf = x.reshape(NW, NCHUNK, G).astype(jnp.int32)
    out = _emb(xf, table)
    return out.reshape(b, t, D_MODEL)


# speedup vs baseline: 1.8621x; 1.0019x over previous
import jax
import jax.numpy as jnp
from jax import lax
from jax.experimental import pallas as pl
from jax.experimental.pallas import tpu as pltpu
from jax.experimental.pallas import tpu_sc as plsc

D_MODEL = 128
NC = 2
NS = 16
NW = NC * NS

G = 128
NCHUNK = 200
NB = 5
AHEAD = NB - 1


def _emb_body(x_hbm, table_hbm, out_hbm, idx_v, *scratch):
    rows = scratch[:NB]
    gsem = scratch[NB:2 * NB]
    wsem = scratch[2 * NB:]
    cc = lax.axis_index("c")
    ss = lax.axis_index("s")
    wid = ss * NC + cc
    pltpu.sync_copy(x_hbm.at[wid], idx_v)
    base = wid * (NCHUNK * G)

    def fire_g(ci, b):
        pltpu.async_copy(table_hbm.at[idx_v.at[ci]], rows[b], gsem[b])

    def wait_g(b):
        pltpu.make_async_copy(table_hbm.at[pl.ds(0, G)], rows[b], gsem[b]).wait()

    def fire_w(ci, b):
        pltpu.async_copy(rows[b], out_hbm.at[pl.ds(base + ci * G, G)], wsem[b])

    def wait_w(b):
        pltpu.make_async_copy(rows[b], out_hbm.at[pl.ds(base, G)], wsem[b]).wait()

    for b in range(AHEAD):
        fire_g(b, b)

    def slot(ci, b, skip_w_wait=False, may_fire=True):
        wait_g(b)
        fire_w(ci, b)
        fb = (b + NB - 1) % NB
        if not skip_w_wait:
            wait_w(fb)
        if may_fire:
            fire_g(ci + AHEAD, fb)

    for b in range(NB):
        slot(b, b, skip_w_wait=(b == 0))

    def step(s, carry):
        for b in range(NB):
            slot(s * NB + b, b)
        return carry

    lax.fori_loop(1, NCHUNK // NB - 1, step, 0)

    s = NCHUNK // NB - 1
    for b in range(NB):
        ci = s * NB + b
        slot(ci, b, may_fire=(ci + AHEAD < NCHUNK))
    wait_w((NCHUNK - 1) % NB)


@jax.jit
def _emb(xf, table):
    mesh = plsc.VectorSubcoreMesh(core_axis_name="c", subcore_axis_name="s")
    kern = pl.kernel(
        _emb_body,
        out_type=jax.ShapeDtypeStruct((NW * NCHUNK * G, D_MODEL), jnp.float32),
        mesh=mesh,
        scratch_types=(
            [pltpu.VMEM((NCHUNK, G), jnp.int32)]
            + [pltpu.VMEM((G, D_MODEL), jnp.float32) for _ in range(NB)]
            + [pltpu.SemaphoreType.DMA for _ in range(2 * NB)]
        ),
    )
    return kern(xf, table)


def kernel(x, table):
    b, t = x.shape
    xf = x.reshape(NW, NCHUNK, G).astype(jnp.int32)
    out = _emb(xf, table)
    return out.reshape(b, t, D_MODEL)


# speedup vs baseline: 1.9560x; 1.0505x over previous
import jax
import jax.numpy as jnp
from jax import lax
from jax.experimental import pallas as pl
from jax.experimental.pallas import tpu as pltpu
from jax.experimental.pallas import tpu_sc as plsc

D_MODEL = 128
NC = 2
NS = 16
NW = NC * NS

G = 128
NCHUNK = 200
NB = 4
AHEAD = NB - 1
SPH = 1
RB = 2
NGROUP = NCHUNK // SPH


def _emb_body(x_hbm, table_hbm, out_hbm, idx_v, shr, *scratch):
    rows = scratch[:NB]
    wsem = scratch[NB:2 * NB]
    dsem = scratch[2 * NB:2 * NB + RB]
    gsem = scratch[2 * NB + RB:]
    cc = lax.axis_index("c")
    ss = lax.axis_index("s")
    wid = ss * NC + cc
    pltpu.sync_copy(x_hbm.at[wid], idx_v)
    base = wid * (NCHUNK * G)

    def fire_g(ci, b):
        pltpu.async_copy(table_hbm.at[idx_v.at[ci]], rows[b], gsem[b])

    def wait_g(b):
        pltpu.make_async_copy(table_hbm.at[pl.ds(0, G)], rows[b], gsem[b]).wait()

    def fire_x(b, r):
        pltpu.async_copy(rows[b], shr.at[ss, r], wsem[b])

    def wait_x(b):
        pltpu.make_async_copy(rows[b], shr.at[ss, 0], wsem[b]).wait()

    def fire_d(gi, r):
        pltpu.async_copy(
            shr.at[ss, r], out_hbm.at[pl.ds(base + gi * SPH * G, SPH * G)], dsem[r])

    def wait_d(r):
        pltpu.make_async_copy(
            shr.at[ss, r], out_hbm.at[pl.ds(base, SPH * G)], dsem[r]).wait()

    for b in range(AHEAD):
        fire_g(b, b)

    def make_step(first=False, last=False):
        def step(s, carry):
            for b in range(NB):
                ci = s * NB + b
                r = b % RB
                wait_g(b)
                if not (first and b == 0):
                    wait_x((b + NB - 1) % NB)
                    pltpu.async_copy(
                        shr.at[ss, (r + RB - 1) % RB],
                        out_hbm.at[pl.ds(base + (ci - 1) * G, G)],
                        dsem[(r + RB - 1) % RB])
                if (not first) or b >= RB:
                    wait_d(r)
                fire_x(b, r)
                if (not last) or (ci + AHEAD < NCHUNK):
                    fire_g(ci + AHEAD, (b + NB - 1) % NB)
            return carry
        return step

    make_step(first=True)(0, 0)
    lax.fori_loop(1, NCHUNK // NB - 1, make_step(), 0)
    make_step(last=True)(NCHUNK // NB - 1, 0)

    wait_x((NCHUNK - 1) % NB)
    fire_d(NGROUP - 1, (NGROUP - 1) % RB)
    wait_d((NGROUP - 2) % RB)
    wait_d((NGROUP - 1) % RB)


@jax.jit
def _emb(xf, table):
    mesh = plsc.VectorSubcoreMesh(core_axis_name="c", subcore_axis_name="s")
    kern = pl.kernel(
        _emb_body,
        out_type=jax.ShapeDtypeStruct((NW * NCHUNK * G, D_MODEL), jnp.float32),
        mesh=mesh,
        scratch_types=(
            [pltpu.VMEM((NCHUNK, G), jnp.int32),
             pltpu.VMEM_SHARED((NS, RB, SPH * G, D_MODEL), jnp.float32)]
            + [pltpu.VMEM((G, D_MODEL), jnp.float32) for _ in range(NB)]
            + [pltpu.SemaphoreType.DMA for _ in range(NB + RB + NB)]
        ),
    )
    return kern(xf, table)


def kernel(x, table):
    b, t = x.shape
    xf = x.reshape(NW, NCHUNK, G).astype(jnp.int32)
    out = _emb(xf, table)
    return out.reshape(b, t, D_MODEL)
